# baseline (speedup 1.0000x reference)
import functools

import jax
import jax.numpy as jnp
from jax import lax
from jax.experimental import pallas as pl
from jax.experimental.pallas import tpu as pltpu
from jax.experimental.pallas import tpu_sc as plsc

E = 320000
C = 40000
S = 7
D = 128
CS = C * S

NC_SC = 2
NS_SC = 16
NW = NC_SC * NS_SC

G_GATHER = 112
N_GCHUNK = CS // G_GATHER
G_ITERS = (N_GCHUNK + NW - 1) // NW


def _sc_gather_body(edge_hbm, pairs_hbm, out_hbm, pair_v, i0_v, i1_v, rows_v,
                    sem):
  cid = lax.axis_index("c")
  tid = lax.axis_index("s")
  wid = tid * NC_SC + cid
  lanes2 = lax.iota(jnp.int32, 16) * 2

  def chunk(k, carry):
    c = wid + k * NW

    @pl.when(c < N_GCHUNK)
    def _():
      base = c * G_GATHER
      pltpu.sync_copy(pairs_hbm.at[pl.ds(base * 2, G_GATHER * 2)], pair_v)
      for q in range(G_GATHER // 16):
        i0_v[pl.ds(q * 16, 16)] = plsc.load_gather(pair_v, [lanes2 + q * 32])
        i1_v[pl.ds(q * 16, 16)] = plsc.load_gather(pair_v,
                                                   [lanes2 + (q * 32 + 1)])
      pltpu.async_copy(edge_hbm.at[i0_v], rows_v, sem).wait()
      pltpu.async_copy(edge_hbm.at[i1_v], rows_v, sem, add=True).wait()
      pltpu.sync_copy(rows_v, out_hbm.at[pl.ds(base, G_GATHER)])

    return carry

  lax.fori_loop(0, G_ITERS, chunk, 0)


@functools.cache
def _sc_gather():
  return pl.kernel(
      _sc_gather_body,
      out_type=jax.ShapeDtypeStruct((CS, D), jnp.float32),
      mesh=plsc.VectorSubcoreMesh(core_axis_name="c", subcore_axis_name="s"),
      compiler_params=pltpu.CompilerParams(needs_layout_passes=False),
      scratch_types=[
          pltpu.VMEM((G_GATHER * 2,), jnp.int32),
          pltpu.VMEM((G_GATHER,), jnp.int32),
          pltpu.VMEM((G_GATHER,), jnp.int32),
          pltpu.VMEM((G_GATHER, D), jnp.float32),
          pltpu.SemaphoreType.DMA,
      ],
  )


N_CHUNK = 32
CHR = 10240
SPROWS = CHR + 256
SEG = 2800
NSEG = CS // SEG
SEGS_PER_TILE = 7
SEG_GROUPS = SEG // 16
L_CAP = 2176
LIST_ROWS = 40


HALF_CHUNKS = 16
H0_ROWS = HALF_CHUNKS * CHR
H1_ROWS = E - H0_ROWS


def _sc_scatter_body(chunk_base, rows_hbm, idx_hbm, zeros_hbm, out_hbm,
                     segbuf, loff_list, slot_list, rows_v, cnt_tmp, spmem,
                     sem, sems):
  cid = lax.axis_index("c")
  tid = lax.axis_index("s")

  lanes = lax.iota(jnp.int32, 16)
  dummy_row = CHR + tid * 16
  pad_slot = tid * SEG

  def list_store(pos, loff, slots, m):
    plsc.store_scatter(loff_list, [pos >> 7, pos & 127], loff, mask=m)
    plsc.store_scatter(slot_list, [pos >> 7, pos & 127], slots, mask=m)

  def gather_start(b):
    pltpu.async_copy(rows_hbm.at[slot_list.at[b]], rows_v.at[b % 2], sem)

  def gather_wait(b):
    pltpu.make_async_copy(rows_hbm.at[slot_list.at[b]], rows_v.at[b % 2],
                          sem).wait()

  def scat_start(b):
    pltpu.async_copy(rows_v.at[b % 2], spmem.at[loff_list.at[b]], sems,
                     add=True)

  def scat_wait(b):
    pltpu.make_async_copy(rows_v.at[b % 2], spmem.at[loff_list.at[b]],
                          sems).wait()

  def flush(cnt):
    dfill = jnp.full((16,), dummy_row, jnp.int32)
    sfill = jnp.full((16,), pad_slot, jnp.int32)
    for p in range(8):
      pad = cnt + p * 16 + lanes
      list_store(pad, dfill, sfill, pad < LIST_ROWS * 128)
    nb = (cnt + 127) // 128
    gather_start(0)

    def batch(b, carry2):
      gather_wait(b)
      scat_start(b)

      @pl.when(b >= 1)
      def _():
        scat_wait(b - 1)

      @pl.when(b + 1 < nb)
      def _():
        gather_start(b + 1)

      return carry2

    lax.fori_loop(0, nb, batch, 0)
    scat_wait(nb - 1)

  def per_chunk(ci, carry):
    chunk = chunk_base + 2 * ci + cid
    lo = chunk * CHR

    pltpu.sync_copy(zeros_hbm, rows_v.at[0])
    for q in range(5):
      pltpu.sync_copy(rows_v.at[0], spmem.at[pl.ds(tid * 656 + q * 128, 128)])
    pltpu.sync_copy(zeros_hbm.at[pl.ds(0, 16)],
                    spmem.at[pl.ds(tid * 656 + 640, 16)])
    plsc.subcore_barrier()

    cnt_vec = jnp.zeros((16,), jnp.int32)
    for k in range(SEGS_PER_TILE):
      seg_id = tid + k * NS_SC
      seg_ok = seg_id < NSEG
      off = jnp.minimum(seg_id, NSEG - 1) * SEG
      pltpu.sync_copy(idx_hbm.at[pl.ds(pl.multiple_of(off, 8), SEG)], segbuf)

      def scan_group(g, cnt):
        v = segbuf[pl.ds(pl.multiple_of(g * 16, 16), 16)]
        m = (v >= lo) & (v < lo + CHR) & seg_ok
        ones = m.astype(jnp.int32)
        cum = plsc.cumsum(ones)
        suf = lax.rev(plsc.cumsum(lax.rev(ones, (0,))), (0,))
        pos = jnp.maximum(cnt + cum - 1, 0)
        list_store(pos, v - lo, off + g * 16 + lanes, m)
        return cnt + (cum + suf - ones)

      cnt_vec = lax.fori_loop(0, SEG_GROUPS, scan_group, cnt_vec)

      cnt_tmp[pl.ds(0, 16)] = cnt_vec
      cs = cnt_tmp[pl.ds(0, 16)][0]
      due = (cs >= L_CAP) if k < SEGS_PER_TILE - 1 else (cs > 0)

      @pl.when(due)
      def _():
        flush(cs)

      cnt_vec = jnp.where(due, 0, cnt_vec)

    plsc.subcore_barrier()

    @pl.when(lo + tid * 640 < E)
    def _():
      pltpu.sync_copy(
          spmem.at[pl.ds(tid * 640, 640)],
          out_hbm.at[pl.ds(lo - chunk_base * CHR + tid * 640, 640)])

    plsc.subcore_barrier()
    return carry

  lax.fori_loop(0, HALF_CHUNKS // NC_SC, per_chunk, 0)


@functools.cache
def _sc_scatter(chunk_base):
  return pl.kernel(
      functools.partial(_sc_scatter_body, chunk_base),
      out_type=jax.ShapeDtypeStruct(
          (H0_ROWS if chunk_base == 0 else H1_ROWS, D), jnp.float32),
      mesh=plsc.VectorSubcoreMesh(core_axis_name="c", subcore_axis_name="s"),
      compiler_params=pltpu.CompilerParams(needs_layout_passes=False),
      scratch_types=[
          pltpu.VMEM((SEG,), jnp.int32),
          pltpu.VMEM((LIST_ROWS, 128), jnp.int32),
          pltpu.VMEM((LIST_ROWS, 128), jnp.int32),
          pltpu.VMEM((2, 128, D), jnp.float32),
          pltpu.VMEM((16,), jnp.int32),
          pltpu.VMEM_SHARED((SPROWS, D), jnp.float32),
          pltpu.SemaphoreType.DMA,
          pltpu.SemaphoreType.DMA,
      ],
  )


B_CYC = 1000


def _mm(x, w_ref):
  return jnp.dot(x.astype(jnp.bfloat16), w_ref[...],
                 preferred_element_type=jnp.float32)


def _tc_cycle_body(e2c_ref, br_ref, a_ref, w1a_ref, w1b_ref, w1c_ref, wc_ref,
                   b1_ref, w2_ref, b2_ref, w3_ref, b3_ref, out_ref):
  xs = [e2c_ref[:, j * D:(j + 1) * D] for j in range(S)]
  rs = xs[0]
  for j in range(1, S):
    rs = rs + xs[j]
  base = _mm(rs, w1c_ref) + b1_ref[...]
  for s in range(S):
    t = a_ref[s, 0] * xs[0]
    for j in range(1, S):
      t = t + a_ref[s, j] * xs[j]
    h = (base
         + _mm(br_ref[:, s * D:(s + 1) * D], w1a_ref)
         + _mm(xs[s], w1b_ref)
         + _mm(t, wc_ref))
    h = jnp.maximum(h, 0.0)
    h = jnp.maximum(_mm(h, w2_ref) + b2_ref[...], 0.0)
    out_ref[:, s * D:(s + 1) * D] = _mm(h, w3_ref) + b3_ref[...]


def _tc_cycle(e2c, br, a_aut, w1a, w1b, w1c, wc, b1e, w2, b2, w3, b3):
  h2 = 2 * D
  grid = (C // B_CYC,)
  full = lambda shape: pl.BlockSpec(shape, lambda i: (0, 0))
  return pl.pallas_call(
      _tc_cycle_body,
      grid=grid,
      in_specs=[
          pl.BlockSpec((B_CYC, S * D), lambda i: (i, 0)),
          pl.BlockSpec((B_CYC, S * D), lambda i: (i, 0)),
          pl.BlockSpec(memory_space=pltpu.SMEM),
          full((D, h2)), full((D, h2)), full((D, h2)), full((D, h2)),
          full((1, h2)), full((h2, h2)), full((1, h2)),
          full((h2, D)), full((1, D)),
      ],
      out_specs=pl.BlockSpec((B_CYC, S * D), lambda i: (i, 0)),
      out_shape=jax.ShapeDtypeStruct((C, S * D), jnp.float32),
  )(e2c, br, a_aut, w1a, w1b, w1c, wc, b1e, w2, b2, w3, b3)


B_EDG = 2560


def _tc_edge_body(er_ref, c2e_ref, we1a_ref, we1b_ref, be1_ref, we2_ref,
                  be2_ref, out_ref):
  h = _mm(er_ref[...], we1a_ref) + _mm(c2e_ref[...], we1b_ref) + be1_ref[...]
  h = jnp.maximum(h, 0.0)
  out_ref[...] = _mm(h, we2_ref) + be2_ref[...]


def _tc_edge(er, c2e, we1a, we1b, be1, we2, be2, row0):
  h2 = 2 * D
  n_rows = c2e.shape[0]
  blk0 = row0 // B_EDG
  full = lambda shape: pl.BlockSpec(shape, lambda i: (0, 0))
  return pl.pallas_call(
      _tc_edge_body,
      grid=(n_rows // B_EDG,),
      in_specs=[
          pl.BlockSpec((B_EDG, D), lambda i: (i + blk0, 0)),
          pl.BlockSpec((B_EDG, D), lambda i: (i, 0)),
          full((D, h2)), full((D, h2)), full((1, h2)),
          full((h2, D)), full((1, D)),
      ],
      out_specs=pl.BlockSpec((B_EDG, D), lambda i: (i, 0)),
      out_shape=jax.ShapeDtypeStruct((n_rows, D), jnp.float32),
  )(er, c2e, we1a, we1b, be1, we2, be2)


def kernel(edge_rep, br_cycle_rep, edge2cycle_idx, cycle2edge_idx, A_aut,
           W_aut, b_aut, W1, b1, W2, b2, W3, b3, We1, be1, We2, be2):
  one = jax.lax.optimization_barrier(jnp.int32(1))
  pairs = edge2cycle_idx.reshape(CS * 2).astype(jnp.int32) * one
  e2c = _sc_gather()(edge_rep, pairs)

  bf16 = jnp.bfloat16
  w1a = W1[0:D].astype(bf16)
  w1b = W1[D:2 * D].astype(bf16)
  w1c = W1[2 * D:3 * D].astype(bf16)
  w1d = W1[3 * D:]
  wc = (W_aut @ w1d).astype(bf16)
  b1e = (b1 + b_aut @ w1d).reshape(1, -1)

  br_flat = br_cycle_rep.reshape(C, S * D)
  br_out = _tc_cycle(e2c.reshape(C, S * D), br_flat, A_aut, w1a, w1b, w1c,
                     wc, b1e, W2.astype(bf16), b2.reshape(1, -1),
                     W3.astype(bf16), b3.reshape(1, -1))

  c2e_idx = cycle2edge_idx.reshape(CS).astype(jnp.int32) * one
  zeros = jnp.zeros((128, D), jnp.float32)
  br_rows = br_out.reshape(CS, D)
  c2e0 = _sc_scatter(0)(br_rows, c2e_idx, zeros)
  c2e1 = _sc_scatter(HALF_CHUNKS)(br_rows, c2e_idx, zeros)

  ew = (We1[0:D].astype(bf16), We1[D:].astype(bf16), be1.reshape(1, -1),
        We2.astype(bf16), be2.reshape(1, -1))
  e0 = _tc_edge(edge_rep, c2e0, *ew, 0)
  e1 = _tc_edge(edge_rep, c2e1, *ew, H0_ROWS)
  edge_out = jnp.concatenate([e0, e1], axis=0)
  return (edge_out, br_out.reshape(C, S, D))


# speedup vs baseline: 1.0180x; 1.0180x over previous
import functools

import jax
import jax.numpy as jnp
from jax import lax
from jax.experimental import pallas as pl
from jax.experimental.pallas import tpu as pltpu
from jax.experimental.pallas import tpu_sc as plsc

E = 320000
C = 40000
S = 7
D = 128
CS = C * S

NC_SC = 2
NS_SC = 16
NW = NC_SC * NS_SC

G_GATHER = 112
N_GCHUNK = CS // G_GATHER
G_ITERS = (N_GCHUNK + NW - 1) // NW


def _sc_gather_body(edge_hbm, pairs_hbm, out_hbm, pair_v, i0_v, i1_v, rows_v,
                    sem):
  cid = lax.axis_index("c")
  tid = lax.axis_index("s")
  wid = tid * NC_SC + cid
  lanes2 = lax.iota(jnp.int32, 16) * 2

  def chunk(k, carry):
    c = wid + k * NW

    @pl.when(c < N_GCHUNK)
    def _():
      base = c * G_GATHER
      pltpu.sync_copy(pairs_hbm.at[pl.ds(base * 2, G_GATHER * 2)], pair_v)
      for q in range(G_GATHER // 16):
        i0_v[pl.ds(q * 16, 16)] = plsc.load_gather(pair_v, [lanes2 + q * 32])
        i1_v[pl.ds(q * 16, 16)] = plsc.load_gather(pair_v,
                                                   [lanes2 + (q * 32 + 1)])
      pltpu.async_copy(edge_hbm.at[i0_v], rows_v, sem).wait()
      pltpu.async_copy(edge_hbm.at[i1_v], rows_v, sem, add=True).wait()
      pltpu.sync_copy(rows_v, out_hbm.at[pl.ds(base, G_GATHER)])

    return carry

  lax.fori_loop(0, G_ITERS, chunk, 0)


@functools.cache
def _sc_gather():
  return pl.kernel(
      _sc_gather_body,
      out_type=jax.ShapeDtypeStruct((CS, D), jnp.float32),
      mesh=plsc.VectorSubcoreMesh(core_axis_name="c", subcore_axis_name="s"),
      compiler_params=pltpu.CompilerParams(needs_layout_passes=False),
      scratch_types=[
          pltpu.VMEM((G_GATHER * 2,), jnp.int32),
          pltpu.VMEM((G_GATHER,), jnp.int32),
          pltpu.VMEM((G_GATHER,), jnp.int32),
          pltpu.VMEM((G_GATHER, D), jnp.float32),
          pltpu.SemaphoreType.DMA,
      ],
  )


N_CHUNK = 32
CHR = 10240
SPROWS = CHR + 256
SEG = 2800
NSEG = CS // SEG
SEGS_PER_TILE = 7
SEG_GROUPS = SEG // 16
L_CAP = 2176
LIST_ROWS = 40


HALF_CHUNKS = N_CHUNK


def _sc_scatter_body(chunk_base, rows_hbm, idx_hbm, zeros_hbm, out_hbm,
                     segbuf, loff_list, slot_list, rows_v, cnt_tmp, spmem,
                     sem, sems):
  cid = lax.axis_index("c")
  tid = lax.axis_index("s")

  lanes = lax.iota(jnp.int32, 16)
  dummy_row = CHR + tid * 16
  pad_slot = tid * SEG

  def list_store(pos, loff, slots, m):
    plsc.store_scatter(loff_list, [pos >> 7, pos & 127], loff, mask=m)
    plsc.store_scatter(slot_list, [pos >> 7, pos & 127], slots, mask=m)

  def gather_start(b):
    pltpu.async_copy(rows_hbm.at[slot_list.at[b]], rows_v.at[b % 2], sem)

  def gather_wait(b):
    pltpu.make_async_copy(rows_hbm.at[slot_list.at[b]], rows_v.at[b % 2],
                          sem).wait()

  def scat_start(b):
    pltpu.async_copy(rows_v.at[b % 2], spmem.at[loff_list.at[b]], sems,
                     add=True)

  def scat_wait(b):
    pltpu.make_async_copy(rows_v.at[b % 2], spmem.at[loff_list.at[b]],
                          sems).wait()

  def flush(cnt):
    dfill = jnp.full((16,), dummy_row, jnp.int32)
    sfill = jnp.full((16,), pad_slot, jnp.int32)
    for p in range(8):
      pad = cnt + p * 16 + lanes
      list_store(pad, dfill, sfill, pad < LIST_ROWS * 128)
    nb = (cnt + 127) // 128
    gather_start(0)

    def batch(b, carry2):
      gather_wait(b)
      scat_start(b)

      @pl.when(b >= 1)
      def _():
        scat_wait(b - 1)

      @pl.when(b + 1 < nb)
      def _():
        gather_start(b + 1)

      return carry2

    lax.fori_loop(0, nb, batch, 0)
    scat_wait(nb - 1)

  def per_chunk(ci, carry):
    chunk = chunk_base + 2 * ci + cid
    lo = chunk * CHR

    pltpu.sync_copy(zeros_hbm, rows_v.at[0])
    for q in range(5):
      pltpu.sync_copy(rows_v.at[0], spmem.at[pl.ds(tid * 656 + q * 128, 128)])
    pltpu.sync_copy(zeros_hbm.at[pl.ds(0, 16)],
                    spmem.at[pl.ds(tid * 656 + 640, 16)])
    plsc.subcore_barrier()

    cnt_vec = jnp.zeros((16,), jnp.int32)
    for k in range(SEGS_PER_TILE):
      seg_id = tid + k * NS_SC
      seg_ok = seg_id < NSEG
      off = jnp.minimum(seg_id, NSEG - 1) * SEG
      pltpu.sync_copy(idx_hbm.at[pl.ds(pl.multiple_of(off, 8), SEG)], segbuf)

      def scan_group(g, cnt):
        v = segbuf[pl.ds(pl.multiple_of(g * 16, 16), 16)]
        m = (v >= lo) & (v < lo + CHR) & seg_ok
        ones = m.astype(jnp.int32)
        cum = plsc.cumsum(ones)
        suf = lax.rev(plsc.cumsum(lax.rev(ones, (0,))), (0,))
        pos = jnp.maximum(cnt + cum - 1, 0)
        list_store(pos, v - lo, off + g * 16 + lanes, m)
        return cnt + (cum + suf - ones)

      cnt_vec = lax.fori_loop(0, SEG_GROUPS, scan_group, cnt_vec)

      cnt_tmp[pl.ds(0, 16)] = cnt_vec
      cs = cnt_tmp[pl.ds(0, 16)][0]
      due = (cs >= L_CAP) if k < SEGS_PER_TILE - 1 else (cs > 0)

      @pl.when(due)
      def _():
        flush(cs)

      cnt_vec = jnp.where(due, 0, cnt_vec)

    plsc.subcore_barrier()

    @pl.when(lo + tid * 640 < E)
    def _():
      pltpu.sync_copy(
          spmem.at[pl.ds(tid * 640, 640)],
          out_hbm.at[pl.ds(lo - chunk_base * CHR + tid * 640, 640)])

    plsc.subcore_barrier()
    return carry

  lax.fori_loop(0, HALF_CHUNKS // NC_SC, per_chunk, 0)


@functools.cache
def _sc_scatter(chunk_base):
  return pl.kernel(
      functools.partial(_sc_scatter_body, chunk_base),
      out_type=jax.ShapeDtypeStruct((E, D), jnp.float32),
      mesh=plsc.VectorSubcoreMesh(core_axis_name="c", subcore_axis_name="s"),
      compiler_params=pltpu.CompilerParams(needs_layout_passes=False),
      scratch_types=[
          pltpu.VMEM((SEG,), jnp.int32),
          pltpu.VMEM((LIST_ROWS, 128), jnp.int32),
          pltpu.VMEM((LIST_ROWS, 128), jnp.int32),
          pltpu.VMEM((2, 128, D), jnp.float32),
          pltpu.VMEM((16,), jnp.int32),
          pltpu.VMEM_SHARED((SPROWS, D), jnp.float32),
          pltpu.SemaphoreType.DMA,
          pltpu.SemaphoreType.DMA,
      ],
  )


B_CYC = 1000


def _mm(x, w_ref):
  return jnp.dot(x.astype(jnp.bfloat16), w_ref[...],
                 preferred_element_type=jnp.float32)


def _tc_cycle_body(e2c_ref, br_ref, a_ref, w1a_ref, w1b_ref, w1c_ref, wc_ref,
                   b1_ref, w2_ref, b2_ref, w3_ref, b3_ref, out_ref):
  xs = [e2c_ref[:, j * D:(j + 1) * D] for j in range(S)]
  rs = xs[0]
  for j in range(1, S):
    rs = rs + xs[j]
  base = _mm(rs, w1c_ref) + b1_ref[...]
  for s in range(S):
    t = a_ref[s, 0] * xs[0]
    for j in range(1, S):
      t = t + a_ref[s, j] * xs[j]
    h = (base
         + _mm(br_ref[:, s * D:(s + 1) * D], w1a_ref)
         + _mm(xs[s], w1b_ref)
         + _mm(t, wc_ref))
    h = jnp.maximum(h, 0.0)
    h = jnp.maximum(_mm(h, w2_ref) + b2_ref[...], 0.0)
    out_ref[:, s * D:(s + 1) * D] = _mm(h, w3_ref) + b3_ref[...]


def _tc_cycle(e2c, br, a_aut, w1a, w1b, w1c, wc, b1e, w2, b2, w3, b3):
  h2 = 2 * D
  grid = (C // B_CYC,)
  full = lambda shape: pl.BlockSpec(shape, lambda i: (0, 0))
  return pl.pallas_call(
      _tc_cycle_body,
      grid=grid,
      in_specs=[
          pl.BlockSpec((B_CYC, S * D), lambda i: (i, 0)),
          pl.BlockSpec((B_CYC, S * D), lambda i: (i, 0)),
          pl.BlockSpec(memory_space=pltpu.SMEM),
          full((D, h2)), full((D, h2)), full((D, h2)), full((D, h2)),
          full((1, h2)), full((h2, h2)), full((1, h2)),
          full((h2, D)), full((1, D)),
      ],
      out_specs=pl.BlockSpec((B_CYC, S * D), lambda i: (i, 0)),
      out_shape=jax.ShapeDtypeStruct((C, S * D), jnp.float32),
  )(e2c, br, a_aut, w1a, w1b, w1c, wc, b1e, w2, b2, w3, b3)


B_EDG = 2560


def _tc_edge_body(er_ref, c2e_ref, we1a_ref, we1b_ref, be1_ref, we2_ref,
                  be2_ref, out_ref):
  h = _mm(er_ref[...], we1a_ref) + _mm(c2e_ref[...], we1b_ref) + be1_ref[...]
  h = jnp.maximum(h, 0.0)
  out_ref[...] = _mm(h, we2_ref) + be2_ref[...]


def _tc_edge(er, c2e, we1a, we1b, be1, we2, be2, row0):
  h2 = 2 * D
  n_rows = c2e.shape[0]
  blk0 = row0 // B_EDG
  full = lambda shape: pl.BlockSpec(shape, lambda i: (0, 0))
  return pl.pallas_call(
      _tc_edge_body,
      grid=(n_rows // B_EDG,),
      in_specs=[
          pl.BlockSpec((B_EDG, D), lambda i: (i + blk0, 0)),
          pl.BlockSpec((B_EDG, D), lambda i: (i, 0)),
          full((D, h2)), full((D, h2)), full((1, h2)),
          full((h2, D)), full((1, D)),
      ],
      out_specs=pl.BlockSpec((B_EDG, D), lambda i: (i, 0)),
      out_shape=jax.ShapeDtypeStruct((n_rows, D), jnp.float32),
  )(er, c2e, we1a, we1b, be1, we2, be2)


def kernel(edge_rep, br_cycle_rep, edge2cycle_idx, cycle2edge_idx, A_aut,
           W_aut, b_aut, W1, b1, W2, b2, W3, b3, We1, be1, We2, be2):
  one = jax.lax.optimization_barrier(jnp.int32(1))
  pairs = edge2cycle_idx.reshape(CS * 2).astype(jnp.int32) * one
  e2c = _sc_gather()(edge_rep, pairs)

  bf16 = jnp.bfloat16
  w1a = W1[0:D].astype(bf16)
  w1b = W1[D:2 * D].astype(bf16)
  w1c = W1[2 * D:3 * D].astype(bf16)
  w1d = W1[3 * D:]
  wc = (W_aut @ w1d).astype(bf16)
  b1e = (b1 + b_aut @ w1d).reshape(1, -1)

  br_flat = br_cycle_rep.reshape(C, S * D)
  br_out = _tc_cycle(e2c.reshape(C, S * D), br_flat, A_aut, w1a, w1b, w1c,
                     wc, b1e, W2.astype(bf16), b2.reshape(1, -1),
                     W3.astype(bf16), b3.reshape(1, -1))

  c2e_idx = cycle2edge_idx.reshape(CS).astype(jnp.int32) * one
  zeros = jnp.zeros((128, D), jnp.float32)
  c2e = _sc_scatter(0)(br_out.reshape(CS, D), c2e_idx, zeros)

  edge_out = _tc_edge(edge_rep, c2e, We1[0:D].astype(bf16),
                      We1[D:].astype(bf16), be1.reshape(1, -1),
                      We2.astype(bf16), be2.reshape(1, -1), 0)
  return (edge_out, br_out.reshape(C, S, D))
